# baseline (speedup 1.0000x reference)
import functools

import jax
import jax.numpy as jnp
from jax import lax
from jax.experimental import pallas as pl
from jax.experimental.pallas import tpu as pltpu
from jax.experimental.pallas import tpu_sc as plsc

NC = 2
NS = 16
LANES = 16
HEADS = 8
HH = HEADS // NC
HD = 16
HIDDEN = 128
QC = HH * HD
SCALE = float(HD) ** -0.5
B = 128
ACCW = 72


def _proj_body(h_ref, wqt_ref, bq_ref, wkt_ref, bk_ref, wvt_ref, bv_ref,
               q2_ref, kv2_ref):
    hb = h_ref[...]
    q = jnp.dot(hb, wqt_ref[0], preferred_element_type=jnp.float32)
    q2_ref[0] = (q + bq_ref[0]) * SCALE
    k = jnp.dot(hb, wkt_ref[0], preferred_element_type=jnp.float32) + bk_ref[0]
    v = jnp.dot(hb, wvt_ref[0], preferred_element_type=jnp.float32) + bv_ref[0]
    kv2_ref[0] = jnp.concatenate([k, v], axis=1)


def _final_body(o0_ref, o1_ref, d0_ref, d1_ref, sel_ref, wot_ref, bo_ref,
                out_ref):
    p = jnp.concatenate([o0_ref[...], o1_ref[...]], axis=1)
    d = jnp.concatenate([d0_ref[...], d1_ref[...]], axis=1)
    r = jnp.where(d > 0.0, 1.0 / jnp.where(d > 0.0, d, 1.0), 0.0)
    r128 = jnp.dot(r, sel_ref[...], preferred_element_type=jnp.float32)
    out = p * r128
    out_ref[...] = (
        jnp.dot(out, wot_ref[...], preferred_element_type=jnp.float32)
        + bo_ref[...])


def _sc_body(n_nodes, ch, q0_hbm, q1_hbm, kv0_hbm, kv1_hbm,
             rowg_hbm, rowsc_hbm, colg_hbm, zo_hbm, out_hbm, oacc,
             rg0, rg1, rs0, rs1, cg0, cg1,
             qbuf0, qbuf1, kvbuf0, kvbuf1, wvbuf0, wvbuf1,
             gsem0, gsem1, ssem0, ssem1):
    c = lax.axis_index("c")
    s = lax.axis_index("s")
    rpt = n_nodes // NS
    tbase = s * (ch * B)

    pltpu.sync_copy(zo_hbm.at[pl.ds(s * rpt, rpt)], oacc.at[pl.ds(s * rpt, rpt)])
    plsc.subcore_barrier()

    lane_iota = lax.iota(jnp.int32, LANES)
    gbufs = ((qbuf0, kvbuf0, wvbuf0, rg0, rs0, cg0, gsem0, ssem0),
             (qbuf1, kvbuf1, wvbuf1, rg1, rs1, cg1, gsem1, ssem1))

    def issue_gathers(ci, p):
        qb, kb, _, rg, rs, cg, gs, _ = gbufs[p]
        base = tbase + ci * B
        pltpu.sync_copy(rowg_hbm.at[pl.ds(base, B)], rg)
        pltpu.sync_copy(rowsc_hbm.at[pl.ds(base, B)], rs)
        pltpu.sync_copy(colg_hbm.at[pl.ds(base, B)], cg)

        @pl.when(c == 0)
        def _():
            pltpu.async_copy(q0_hbm.at[rg], qb, gs)
            pltpu.async_copy(kv0_hbm.at[cg], kb, gs)

        @pl.when(c == 1)
        def _():
            pltpu.async_copy(q1_hbm.at[rg], qb, gs)
            pltpu.async_copy(kv1_hbm.at[cg], kb, gs)

    def drain_gathers(p):
        qb, kb, _, _, _, _, gs, _ = gbufs[p]
        pltpu.make_async_copy(q0_hbm.at[pl.ds(0, B)], qb, gs).wait()
        pltpu.make_async_copy(kv0_hbm.at[pl.ds(0, B)], kb, gs).wait()

    def issue_scatters(ci, p):
        _, _, wb, _, rs, _, _, ss = gbufs[p]
        pltpu.async_copy(wb, oacc.at[rs], ss, add=True)

    def drain_scatters(p):
        _, _, wb, _, _, _, _, ss = gbufs[p]
        pltpu.make_async_copy(zo_hbm.at[pl.ds(0, B)], wb, ss).wait()

    def compute(p):
        qb, kb, wb = gbufs[p][0], gbufs[p][1], gbufs[p][2]

        def group_body(g, _):
            evec = g * LANES + lane_iota

            def head_body(hh, _):
                cbase = hh * HD
                acc = jnp.zeros((LANES,), jnp.float32)
                for d_ in range(HD):
                    cvec = jnp.full((LANES,), cbase + d_, jnp.int32)
                    qv = plsc.load_gather(qb, [evec, cvec])
                    kv2 = plsc.load_gather(kb, [evec, cvec])
                    acc = acc + qv * kv2
                es = jnp.exp(acc)
                hvec = jnp.full((LANES,), QC + hh, jnp.int32)
                plsc.store_scatter(wb, [evec, hvec], es)
                for d_ in range(HD):
                    cvec = jnp.full((LANES,), cbase + d_, jnp.int32)
                    vv = plsc.load_gather(kb, [evec, QC + cvec])
                    plsc.store_scatter(wb, [evec, cvec], es * vv)
                return 0

            lax.fori_loop(0, HH, head_body, 0)
            return 0

        lax.fori_loop(0, B // LANES, group_body, 0)

    issue_gathers(0, 0)
    npairs = ch // 2

    def pair_body(it, _):
        c0 = 2 * it
        c1 = c0 + 1

        @pl.when(c1 >= 3)
        def _():
            drain_scatters(1)

        issue_gathers(c1, 1)
        drain_gathers(0)

        @pl.when(c0 >= 2)
        def _():
            drain_scatters(0)

        compute(0)
        issue_scatters(c0, 0)

        @pl.when(c0 + 2 < ch)
        def _():
            issue_gathers(c0 + 2, 0)

        drain_gathers(1)
        compute(1)
        issue_scatters(c1, 1)
        return 0

    lax.fori_loop(0, npairs, pair_body, 0)
    drain_scatters(0)
    drain_scatters(1)

    plsc.subcore_barrier()
    pltpu.sync_copy(oacc.at[pl.ds(s * rpt, rpt)],
                    out_hbm.at[c].at[pl.ds(s * rpt, rpt)])


def kernel(h, edge_index, Wq, bq, Wk, bk, Wv, bv, Wo, bo):
    n = h.shape[0]
    e = edge_index.shape[1]
    row = edge_index[0]
    col = edge_index[1]
    ept = e // NS
    ch = ept // B + 2
    pad = ch * B - ept

    bs = 1000
    grid = n // bs

    def _split(wt):
        return wt.reshape(HIDDEN, NC, QC).transpose(1, 0, 2)

    def _bsplit(b):
        return b.reshape(NC, 1, QC)

    q2, kv2 = pl.pallas_call(
        _proj_body,
        grid=(grid, NC),
        in_specs=[
            pl.BlockSpec((bs, HIDDEN), lambda i, c: (i, 0)),
            pl.BlockSpec((1, HIDDEN, QC), lambda i, c: (c, 0, 0)),
            pl.BlockSpec((1, 1, QC), lambda i, c: (c, 0, 0)),
            pl.BlockSpec((1, HIDDEN, QC), lambda i, c: (c, 0, 0)),
            pl.BlockSpec((1, 1, QC), lambda i, c: (c, 0, 0)),
            pl.BlockSpec((1, HIDDEN, QC), lambda i, c: (c, 0, 0)),
            pl.BlockSpec((1, 1, QC), lambda i, c: (c, 0, 0)),
        ],
        out_specs=[
            pl.BlockSpec((1, bs, QC), lambda i, c: (c, i, 0)),
            pl.BlockSpec((1, bs, 2 * QC), lambda i, c: (c, i, 0)),
        ],
        out_shape=[
            jax.ShapeDtypeStruct((NC, n, QC), jnp.float32),
            jax.ShapeDtypeStruct((NC, n, 2 * QC), jnp.float32),
        ],
    )(h, _split(Wq.T), _bsplit(bq), _split(Wk.T), _bsplit(bk),
      _split(Wv.T), _bsplit(bv))

    zo = jnp.zeros((n, ACCW), jnp.float32)

    rowt = row.reshape(NS, ept)
    colt = col.reshape(NS, ept)
    zpad = jnp.zeros((NS, pad), jnp.int32)
    rowg = jnp.concatenate([rowt, zpad], axis=1).reshape(-1)
    rowsc = jnp.concatenate([rowt, zpad + n], axis=1).reshape(-1)
    colg = jnp.concatenate([colt, zpad], axis=1).reshape(-1)

    mesh = plsc.VectorSubcoreMesh(core_axis_name="c", subcore_axis_name="s")
    owes = pl.kernel(
        functools.partial(_sc_body, n, ch),
        out_type=jax.ShapeDtypeStruct((NC, n, ACCW), jnp.float32),
        mesh=mesh,
        scratch_types=[
            pltpu.VMEM_SHARED((n + 8, ACCW), jnp.float32),
            pltpu.VMEM((B,), jnp.int32),
            pltpu.VMEM((B,), jnp.int32),
            pltpu.VMEM((B,), jnp.int32),
            pltpu.VMEM((B,), jnp.int32),
            pltpu.VMEM((B,), jnp.int32),
            pltpu.VMEM((B,), jnp.int32),
            pltpu.VMEM((B, QC), jnp.float32),
            pltpu.VMEM((B, QC), jnp.float32),
            pltpu.VMEM((B, 2 * QC), jnp.float32),
            pltpu.VMEM((B, 2 * QC), jnp.float32),
            pltpu.VMEM((B, ACCW), jnp.float32),
            pltpu.VMEM((B, ACCW), jnp.float32),
            pltpu.SemaphoreType.DMA,
            pltpu.SemaphoreType.DMA,
            pltpu.SemaphoreType.DMA,
            pltpu.SemaphoreType.DMA,
        ],
        compiler_params=pltpu.CompilerParams(use_tc_tiling_on_sc=False,
                                             needs_layout_passes=False),
    )(q2[0], q2[1], kv2[0], kv2[1], rowg, rowsc, colg, zo)

    sel = jnp.kron(jnp.eye(HEADS, dtype=jnp.float32),
                   jnp.ones((1, HD), jnp.float32))

    out = pl.pallas_call(
        _final_body,
        grid=(grid,),
        in_specs=[
            pl.BlockSpec((bs, QC), lambda i: (i, 0)),
            pl.BlockSpec((bs, QC), lambda i: (i, 0)),
            pl.BlockSpec((bs, HH), lambda i: (i, 0)),
            pl.BlockSpec((bs, HH), lambda i: (i, 0)),
            pl.BlockSpec((HEADS, HIDDEN), lambda i: (0, 0)),
            pl.BlockSpec((HIDDEN, HIDDEN), lambda i: (0, 0)),
            pl.BlockSpec((1, HIDDEN), lambda i: (0, 0)),
        ],
        out_specs=pl.BlockSpec((bs, HIDDEN), lambda i: (i, 0)),
        out_shape=jax.ShapeDtypeStruct((n, HIDDEN), jnp.float32),
    )(owes[0, :, :QC], owes[1, :, :QC], owes[0, :, QC:QC + HH], owes[1, :, QC:QC + HH],
      sel, Wo.T, bo.reshape(1, HIDDEN))
    return out


# speedup vs baseline: 1.3150x; 1.3150x over previous
import functools

import jax
import jax.numpy as jnp
from jax import lax
from jax.experimental import pallas as pl
from jax.experimental.pallas import tpu as pltpu
from jax.experimental.pallas import tpu_sc as plsc

NC = 2
NS = 16
NW = NC * NS
LANES = 16
HEADS = 8
HD = 16
HIDDEN = 128
SCALE = float(HD) ** -0.5
B = 80


def _proj_body(h_ref, wqt_ref, bq_ref, wkt_ref, bk_ref, wvt_ref, bv_ref,
               q_ref, kv_ref):
    hb = h_ref[...]
    q = jnp.dot(hb, wqt_ref[...], preferred_element_type=jnp.float32)
    q_ref[...] = (q + bq_ref[...]) * SCALE
    k = jnp.dot(hb, wkt_ref[...], preferred_element_type=jnp.float32) + bk_ref[...]
    v = jnp.dot(hb, wvt_ref[...], preferred_element_type=jnp.float32) + bv_ref[...]
    kv_ref[...] = jnp.concatenate([k, v], axis=1)


def _final_body(o0_ref, o1_ref, d0_ref, d1_ref, sel_ref, wot_ref, bo_ref,
                out_ref):
    p = o0_ref[...] + o1_ref[...]
    d = d0_ref[...] + d1_ref[...]
    r = jnp.where(d > 0.0, 1.0 / jnp.where(d > 0.0, d, 1.0), 0.0)
    r128 = jnp.dot(r, sel_ref[...], preferred_element_type=jnp.float32)
    out = p * r128
    out_ref[...] = (
        jnp.dot(out, wot_ref[...], preferred_element_type=jnp.float32)
        + bo_ref[...])


def _sc_body(n_nodes, epw, q_hbm, kv_hbm, row_hbm, col_hbm, zo_hbm, zd_hbm,
             out_hbm, den_hbm, oacc, dacc, row_v, col_v, qbuf, kvbuf, wvbuf,
             esbuf, sem0, sem1):
    c = lax.axis_index("c")
    s = lax.axis_index("s")
    wid = c * NS + s
    rpt = n_nodes // NS

    pltpu.sync_copy(zo_hbm.at[pl.ds(s * rpt, rpt)], oacc.at[pl.ds(s * rpt, rpt)])
    pltpu.sync_copy(zd_hbm.at[pl.ds(s * rpt, rpt)], dacc.at[pl.ds(s * rpt, rpt)])
    plsc.subcore_barrier()

    lane_iota = lax.iota(jnp.int32, LANES)

    def chunk_body(ci, carry):
        base = wid * epw + ci * B
        pltpu.sync_copy(row_hbm.at[pl.ds(base, B)], row_v)
        pltpu.sync_copy(col_hbm.at[pl.ds(base, B)], col_v)
        cq = pltpu.async_copy(q_hbm.at[row_v], qbuf, sem0)
        ck = pltpu.async_copy(kv_hbm.at[col_v], kvbuf, sem1)
        cq.wait()
        ck.wait()

        def group_body(g, _):
            evec = g * LANES + lane_iota

            def head_body(hh, _):
                cbase = hh * HD
                acc = jnp.zeros((LANES,), jnp.float32)
                for d_ in range(HD):
                    cvec = jnp.full((LANES,), cbase + d_, jnp.int32)
                    qv = plsc.load_gather(qbuf, [evec, cvec])
                    kv2 = plsc.load_gather(kvbuf, [evec, cvec])
                    acc = acc + qv * kv2
                es = jnp.exp(acc)
                hvec = jnp.full((LANES,), hh, jnp.int32)
                plsc.store_scatter(esbuf, [evec, hvec], es)
                for j in range(LANES):
                    e_row = g * LANES + j
                    scv = lax.broadcast_in_dim(es[j], (LANES,), ())
                    vrow = kvbuf[e_row, pl.ds(HIDDEN + cbase, HD)]
                    wvbuf[e_row, pl.ds(cbase, HD)] = vrow * scv
                return 0

            lax.fori_loop(0, HEADS, head_body, 0)
            return 0

        lax.fori_loop(0, B // LANES, group_body, 0)
        pltpu.sync_copy(esbuf, dacc.at[row_v], add=True)
        pltpu.sync_copy(wvbuf, oacc.at[row_v], add=True)
        return carry

    lax.fori_loop(0, epw // B, chunk_body, 0)

    plsc.subcore_barrier()
    pltpu.sync_copy(oacc.at[pl.ds(s * rpt, rpt)],
                    out_hbm.at[c].at[pl.ds(s * rpt, rpt)])
    pltpu.sync_copy(dacc.at[pl.ds(s * rpt, rpt)],
                    den_hbm.at[c].at[pl.ds(s * rpt, rpt)])


def kernel(h, edge_index, Wq, bq, Wk, bk, Wv, bv, Wo, bo):
    n = h.shape[0]
    e = edge_index.shape[1]
    row = edge_index[0]
    col = edge_index[1]
    epw = e // NW

    bs = 1000
    grid = n // bs

    q, kv = pl.pallas_call(
        _proj_body,
        grid=(grid,),
        in_specs=[
            pl.BlockSpec((bs, HIDDEN), lambda i: (i, 0)),
            pl.BlockSpec((HIDDEN, HIDDEN), lambda i: (0, 0)),
            pl.BlockSpec((1, HIDDEN), lambda i: (0, 0)),
            pl.BlockSpec((HIDDEN, HIDDEN), lambda i: (0, 0)),
            pl.BlockSpec((1, HIDDEN), lambda i: (0, 0)),
            pl.BlockSpec((HIDDEN, HIDDEN), lambda i: (0, 0)),
            pl.BlockSpec((1, HIDDEN), lambda i: (0, 0)),
        ],
        out_specs=[
            pl.BlockSpec((bs, HIDDEN), lambda i: (i, 0)),
            pl.BlockSpec((bs, 2 * HIDDEN), lambda i: (i, 0)),
        ],
        out_shape=[
            jax.ShapeDtypeStruct((n, HIDDEN), jnp.float32),
            jax.ShapeDtypeStruct((n, 2 * HIDDEN), jnp.float32),
        ],
    )(h, Wq.T, bq.reshape(1, HIDDEN), Wk.T, bk.reshape(1, HIDDEN),
      Wv.T, bv.reshape(1, HIDDEN))

    zo = jnp.zeros((n, HIDDEN), jnp.float32)
    zd = jnp.zeros((n, HEADS), jnp.float32)

    mesh = plsc.VectorSubcoreMesh(core_axis_name="c", subcore_axis_name="s")
    opart, dpart = pl.kernel(
        functools.partial(_sc_body, n, epw),
        out_type=(
            jax.ShapeDtypeStruct((NC, n, HIDDEN), jnp.float32),
            jax.ShapeDtypeStruct((NC, n, HEADS), jnp.float32),
        ),
        mesh=mesh,
        scratch_types=[
            pltpu.VMEM_SHARED((n, HIDDEN), jnp.float32),
            pltpu.VMEM_SHARED((n, HEADS), jnp.float32),
            pltpu.VMEM((B,), jnp.int32),
            pltpu.VMEM((B,), jnp.int32),
            pltpu.VMEM((B, HIDDEN), jnp.float32),
            pltpu.VMEM((B, 2 * HIDDEN), jnp.float32),
            pltpu.VMEM((B, HIDDEN), jnp.float32),
            pltpu.VMEM((B, HEADS), jnp.float32),
            pltpu.SemaphoreType.DMA,
            pltpu.SemaphoreType.DMA,
        ],
        compiler_params=pltpu.CompilerParams(use_tc_tiling_on_sc=False,
                                             needs_layout_passes=False),
    )(q, kv, row, col, zo, zd)

    sel = jnp.kron(jnp.eye(HEADS, dtype=jnp.float32),
                   jnp.ones((1, HD), jnp.float32))

    out = pl.pallas_call(
        _final_body,
        grid=(grid,),
        in_specs=[
            pl.BlockSpec((bs, HIDDEN), lambda i: (i, 0)),
            pl.BlockSpec((bs, HIDDEN), lambda i: (i, 0)),
            pl.BlockSpec((bs, HEADS), lambda i: (i, 0)),
            pl.BlockSpec((bs, HEADS), lambda i: (i, 0)),
            pl.BlockSpec((HEADS, HIDDEN), lambda i: (0, 0)),
            pl.BlockSpec((HIDDEN, HIDDEN), lambda i: (0, 0)),
            pl.BlockSpec((1, HIDDEN), lambda i: (0, 0)),
        ],
        out_specs=pl.BlockSpec((bs, HIDDEN), lambda i: (i, 0)),
        out_shape=jax.ShapeDtypeStruct((n, HIDDEN), jnp.float32),
    )(opart[0], opart[1], dpart[0], dpart[1], sel, Wo.T,
      bo.reshape(1, HIDDEN))
    return out
